# baseline (speedup 1.0000x reference)
import jax
import jax.numpy as jnp
from jax import lax
from jax.experimental import pallas as pl
from jax.experimental.pallas import tpu as pltpu

DIM = 1024
D_INNER = 1024
HEAD_DIM = 64
NUM_HEADS = D_INNER // HEAD_DIM
D_CONV = 4
TB = 256
CHUNK = 64
HP = jax.lax.Precision.HIGHEST


def _proj_kernel(x_ref, xprev_ref, winT_ref, convw_ref, convb_ref,
                 wqT_ref, wkT_ref, wvT_ref, waT_ref, ba_ref, wbT_ref, bb_ref,
                 q_ref, k_ref, v_ref, a_ref, b_ref):
    i = pl.program_id(1)
    f32 = jnp.float32

    x_cur = x_ref[0]
    xp_cur = jnp.dot(x_cur, winT_ref[...], preferred_element_type=f32,
                     precision=HP)
    xp_prev = jnp.dot(xprev_ref[0, 0], winT_ref[...], preferred_element_type=f32,
                      precision=HP)
    xp_prev = jnp.where(i == 0, jnp.zeros_like(xp_prev), xp_prev)

    ext = jnp.concatenate([xp_prev, xp_cur], axis=0)
    xc = convw_ref[3:4, :] * ext[8:8 + TB]
    xc = xc + convw_ref[2:3, :] * ext[7:7 + TB]
    xc = xc + convw_ref[1:2, :] * ext[6:6 + TB]
    xc = xc + convw_ref[0:1, :] * ext[5:5 + TB]
    xc = xc + convb_ref[...]
    act = xc * jax.nn.sigmoid(xc)

    ridx = lax.broadcasted_iota(jnp.int32, (D_INNER, NUM_HEADS), 0)
    hidx = lax.broadcasted_iota(jnp.int32, (D_INNER, NUM_HEADS), 1)
    seg = (ridx // HEAD_DIM == hidx).astype(f32)

    def _norm(y):
        ss = jnp.dot(y * y, seg, preferred_element_type=f32, precision=HP)
        inv = 1.0 / jnp.maximum(jnp.sqrt(ss), 1e-12)
        inv_full = lax.dot_general(inv, seg, (((1,), (1,)), ((), ())),
                                   preferred_element_type=f32, precision=HP)
        return y * inv_full

    qn = _norm(jnp.dot(act, wqT_ref[...], preferred_element_type=f32,
                       precision=HP))
    kn = _norm(jnp.dot(act, wkT_ref[...], preferred_element_type=f32,
                       precision=HP))
    vv = jnp.dot(act, wvT_ref[...], preferred_element_type=f32, precision=HP)
    for h in range(NUM_HEADS):
        sl = slice(h * HEAD_DIM, (h + 1) * HEAD_DIM)
        q_ref[0, h] = qn[:, sl]
        k_ref[0, h] = kn[:, sl]
        v_ref[0, h] = vv[:, sl]

    asig = jax.nn.sigmoid(jnp.dot(act, waT_ref[...], preferred_element_type=f32,
                                  precision=HP) + ba_ref[...])
    bsig = jax.nn.sigmoid(jnp.dot(act, wbT_ref[...], preferred_element_type=f32,
                                  precision=HP) + bb_ref[...])
    a_ref[0] = jnp.dot(asig, seg, preferred_element_type=f32,
                       precision=HP) * (1.0 / HEAD_DIM)
    b_ref[0] = jnp.dot(bsig, seg, preferred_element_type=f32,
                       precision=HP) * (1.0 / HEAD_DIM)


def _delta_kernel(q_ref, k_ref, v_ref, a_ref, b_ref, woutT_ref,
                  out_ref, sfin_ref, z_ref):
    j = pl.program_id(1)
    f32 = jnp.float32
    C = CHUNK
    NC = 2048 // CHUNK

    @pl.when(j == 0)
    def _init():
        z_ref[...] = jnp.zeros_like(z_ref)

    al = jnp.maximum(a_ref[0], 1e-30)
    be = b_ref[0]

    t_i = lax.broadcasted_iota(jnp.int32, (C, C), 0)
    s_i = lax.broadcasted_iota(jnp.int32, (C, C), 1)
    tril_incl = (t_i >= s_i).astype(f32)
    ell = jnp.dot(tril_incl, jnp.log(al), preferred_element_type=f32, precision=HP)
    eyeC = (t_i == s_i).astype(f32)
    ellT = lax.dot_general(ell, eyeC, (((0,), (0,)), ((), ())),
                           preferred_element_type=f32, precision=HP)

    zall = z_ref[...]
    o_parts = []
    z_parts = []
    for h in range(NUM_HEADS):
        Q = q_ref[0, h]
        K = k_ref[0, h]
        V = v_ref[0, h]
        Kt = K.T
        lc = ell[:, h:h + 1]
        lr = ellT[h:h + 1, :]
        D = jnp.exp(lc - lr)
        bc = be[:, h:h + 1]
        gam = jnp.exp(lc)

        Skk = jnp.dot(K, Kt, preferred_element_type=f32, precision=HP)
        Sqk = jnp.dot(Q, Kt, preferred_element_type=f32, precision=HP)
        G = jnp.where(t_i > s_i, Skk * D, 0.0) * bc
        A = jnp.where(t_i >= s_i, Sqk * D, 0.0)

        Z = zall[h]
        R = (V - jnp.dot(K * gam, Z, preferred_element_type=f32, precision=HP)) * bc
        U = R - jnp.dot(G, R, preferred_element_type=f32, precision=HP)
        Gp = G
        for _ in range(5):
            Gp = jnp.dot(Gp, Gp, preferred_element_type=f32, precision=HP)
            U = U + jnp.dot(Gp, U, preferred_element_type=f32, precision=HP)

        O = (jnp.dot(A, U, preferred_element_type=f32, precision=HP)
             + jnp.dot(Q * gam, Z, preferred_element_type=f32, precision=HP))
        o_parts.append(O)

        gC = jnp.exp(ell[C - 1:C, h:h + 1])
        dec_row = jnp.exp(ell[C - 1:C, h:h + 1] - lr)
        z_parts.append(Z * gC + jnp.dot(Kt * dec_row, U,
                                        preferred_element_type=f32, precision=HP))

    z_ref[...] = jnp.stack(z_parts, axis=0)
    o_full = jnp.concatenate(o_parts, axis=1)
    out_ref[0] = jnp.dot(o_full, woutT_ref[...], preferred_element_type=f32,
                         precision=HP)

    @pl.when(j == NC - 1)
    def _fin():
        for h in range(NUM_HEADS):
            sfin_ref[0, h] = z_ref[h].T


def kernel(x, in_proj_w, conv_w, conv_b, W_q, W_k, W_v, W_alpha, b_alpha,
           W_beta, b_beta, out_proj_w):
    Bsz, T, _ = x.shape
    f32 = jnp.float32
    winT = in_proj_w.T
    convw2 = conv_w[:, 0, :].T
    convb2 = conv_b[None, :]
    wqT, wkT, wvT = W_q.T, W_k.T, W_v.T
    waT, wbT = W_alpha.T, W_beta.T
    ba2, bb2 = b_alpha[None, :], b_beta[None, :]
    woutT = out_proj_w.T
    xh = x.reshape(Bsz, T // 8, 8, DIM)

    nblk = T // TB
    full = lambda b, i: (0, 0)
    H, dh = NUM_HEADS, HEAD_DIM
    q, k, v, al, be = pl.pallas_call(
        _proj_kernel,
        grid=(Bsz, nblk),
        in_specs=[
            pl.BlockSpec((1, TB, DIM), lambda b, i: (b, i, 0)),
            pl.BlockSpec((1, 1, 8, DIM),
                         lambda b, i: (b, jnp.maximum(i * (TB // 8) - 1, 0), 0, 0)),
            pl.BlockSpec((DIM, D_INNER), full),
            pl.BlockSpec((D_CONV, D_INNER), full),
            pl.BlockSpec((1, D_INNER), full),
            pl.BlockSpec((DIM, D_INNER), full),
            pl.BlockSpec((DIM, D_INNER), full),
            pl.BlockSpec((DIM, D_INNER), full),
            pl.BlockSpec((DIM, D_INNER), full),
            pl.BlockSpec((1, D_INNER), full),
            pl.BlockSpec((DIM, D_INNER), full),
            pl.BlockSpec((1, D_INNER), full),
        ],
        out_specs=[
            pl.BlockSpec((1, H, TB, dh), lambda b, i: (b, 0, i, 0)),
            pl.BlockSpec((1, H, TB, dh), lambda b, i: (b, 0, i, 0)),
            pl.BlockSpec((1, H, TB, dh), lambda b, i: (b, 0, i, 0)),
            pl.BlockSpec((1, TB, H), lambda b, i: (b, i, 0)),
            pl.BlockSpec((1, TB, H), lambda b, i: (b, i, 0)),
        ],
        out_shape=[
            jax.ShapeDtypeStruct((Bsz, H, T, dh), f32),
            jax.ShapeDtypeStruct((Bsz, H, T, dh), f32),
            jax.ShapeDtypeStruct((Bsz, H, T, dh), f32),
            jax.ShapeDtypeStruct((Bsz, T, H), f32),
            jax.ShapeDtypeStruct((Bsz, T, H), f32),
        ],
        compiler_params=pltpu.CompilerParams(
            dimension_semantics=("parallel", "arbitrary"),
            vmem_limit_bytes=100 * 1024 * 1024,
        ),
        name="gdn_proj",
    )(x, xh, winT, convw2, convb2, wqT, wkT, wvT, waT, ba2, wbT, bb2)

    nchunk = T // CHUNK
    out, sfin = pl.pallas_call(
        _delta_kernel,
        grid=(Bsz, nchunk),
        in_specs=[
            pl.BlockSpec((1, H, CHUNK, dh), lambda b, j: (b, 0, j, 0)),
            pl.BlockSpec((1, H, CHUNK, dh), lambda b, j: (b, 0, j, 0)),
            pl.BlockSpec((1, H, CHUNK, dh), lambda b, j: (b, 0, j, 0)),
            pl.BlockSpec((1, CHUNK, H), lambda b, j: (b, j, 0)),
            pl.BlockSpec((1, CHUNK, H), lambda b, j: (b, j, 0)),
            pl.BlockSpec((D_INNER, DIM), lambda b, j: (0, 0)),
        ],
        out_specs=[
            pl.BlockSpec((1, CHUNK, DIM), lambda b, j: (b, j, 0)),
            pl.BlockSpec((1, H, dh, dh), lambda b, j: (b, 0, 0, 0)),
        ],
        out_shape=[
            jax.ShapeDtypeStruct((Bsz, T, DIM), f32),
            jax.ShapeDtypeStruct((Bsz, H, dh, dh), f32),
        ],
        scratch_shapes=[pltpu.VMEM((H, dh, dh), f32)],
        compiler_params=pltpu.CompilerParams(
            dimension_semantics=("parallel", "arbitrary"),
            vmem_limit_bytes=100 * 1024 * 1024,
        ),
        name="gdn_delta",
    )(q, k, v, al, be, woutT)

    return out, sfin


# speedup vs baseline: 1.1925x; 1.1925x over previous
import jax
import jax.numpy as jnp
from jax import lax
from jax.experimental import pallas as pl
from jax.experimental.pallas import tpu as pltpu

DIM = 1024
D_INNER = 1024
HEAD_DIM = 64
NUM_HEADS = D_INNER // HEAD_DIM
D_CONV = 4
TB = 256
CHUNK = 64
HP = jax.lax.Precision.HIGHEST
F32 = jnp.float32
BF16 = jnp.bfloat16


def _split(x):
    hi = x.astype(BF16)
    lo = (x - hi.astype(F32)).astype(BF16)
    return hi, lo


def _dot3(x_hi, x_lo, w_hi, w_lo):
    return (jnp.dot(x_hi, w_hi, preferred_element_type=F32)
            + jnp.dot(x_lo, w_hi, preferred_element_type=F32)
            + jnp.dot(x_hi, w_lo, preferred_element_type=F32))


def _proj_kernel(x_ref, xprev_ref, winh_ref, winl_ref, convw_ref, convb_ref,
                 wqh_ref, wql_ref, wkh_ref, wkl_ref, wvh_ref, wvl_ref,
                 wah_ref, wal_ref, ba_ref, wbh_ref, wbl_ref, bb_ref,
                 q_ref, k_ref, v_ref, a_ref, b_ref):
    i = pl.program_id(2)

    xc_hi, xc_lo = _split(x_ref[0])
    xp_cur = _dot3(xc_hi, xc_lo, winh_ref[...], winl_ref[...])
    pv_hi, pv_lo = _split(xprev_ref[0, 0])
    xp_prev = _dot3(pv_hi, pv_lo, winh_ref[...], winl_ref[...])
    xp_prev = jnp.where(i == 0, jnp.zeros_like(xp_prev), xp_prev)

    ext = jnp.concatenate([xp_prev, xp_cur], axis=0)
    xc = convw_ref[3:4, :] * ext[8:8 + TB]
    xc = xc + convw_ref[2:3, :] * ext[7:7 + TB]
    xc = xc + convw_ref[1:2, :] * ext[6:6 + TB]
    xc = xc + convw_ref[0:1, :] * ext[5:5 + TB]
    xc = xc + convb_ref[...]
    act = xc * jax.nn.sigmoid(xc)
    a_hi, a_lo = _split(act)

    ridx = lax.broadcasted_iota(jnp.int32, (D_INNER, NUM_HEADS), 0)
    hidx = lax.broadcasted_iota(jnp.int32, (D_INNER, NUM_HEADS), 1)
    segb = (ridx // HEAD_DIM == hidx).astype(BF16)

    def _segsum(y):
        y_hi, y_lo = _split(y)
        return (jnp.dot(y_hi, segb, preferred_element_type=F32)
                + jnp.dot(y_lo, segb, preferred_element_type=F32))

    def _segexpand(y):
        y_hi, y_lo = _split(y)
        cd = (((1,), (1,)), ((), ()))
        return (lax.dot_general(y_hi, segb, cd, preferred_element_type=F32)
                + lax.dot_general(y_lo, segb, cd, preferred_element_type=F32))

    def _norm(y):
        ss = _segsum(y * y)
        inv = 1.0 / jnp.maximum(jnp.sqrt(ss), 1e-12)
        return y * _segexpand(inv)

    qn = _norm(_dot3(a_hi, a_lo, wqh_ref[...], wql_ref[...]))
    kn = _norm(_dot3(a_hi, a_lo, wkh_ref[...], wkl_ref[...]))
    vv = _dot3(a_hi, a_lo, wvh_ref[...], wvl_ref[...])
    for h in range(NUM_HEADS):
        sl = slice(h * HEAD_DIM, (h + 1) * HEAD_DIM)
        q_ref[0, h] = qn[:, sl]
        k_ref[0, h] = kn[:, sl]
        v_ref[0, h] = vv[:, sl]

    asig = jax.nn.sigmoid(_dot3(a_hi, a_lo, wah_ref[...], wal_ref[...])
                          + ba_ref[...])
    bsig = jax.nn.sigmoid(_dot3(a_hi, a_lo, wbh_ref[...], wbl_ref[...])
                          + bb_ref[...])
    a_ref[0] = _segsum(asig) * (1.0 / HEAD_DIM)
    b_ref[0] = _segsum(bsig) * (1.0 / HEAD_DIM)


def _delta_kernel(q_ref, k_ref, v_ref, a_ref, b_ref, wouth_ref, woutl_ref,
                  out_ref, sfin_ref, z_ref):
    j = pl.program_id(2)
    C = CHUNK
    NC = 2048 // CHUNK

    @pl.when(j == 0)
    def _init():
        z_ref[...] = jnp.zeros_like(z_ref)

    al = jnp.maximum(a_ref[0], 1e-30)
    be = b_ref[0]

    t_i = lax.broadcasted_iota(jnp.int32, (C, C), 0)
    s_i = lax.broadcasted_iota(jnp.int32, (C, C), 1)
    tril_incl = (t_i >= s_i).astype(F32)
    ell = jnp.dot(tril_incl, jnp.log(al), preferred_element_type=F32,
                  precision=HP)
    eyeC = (t_i == s_i).astype(F32)
    ellT = lax.dot_general(ell, eyeC, (((0,), (0,)), ((), ())),
                           preferred_element_type=F32, precision=HP)

    zall = z_ref[...]
    o_parts = []
    z_parts = []
    for h in range(NUM_HEADS):
        Q = q_ref[0, h]
        K = k_ref[0, h]
        V = v_ref[0, h]
        Kt = K.T
        lc = ell[:, h:h + 1]
        lr = ellT[h:h + 1, :]
        D = jnp.exp(lc - lr)
        bc = be[:, h:h + 1]
        gam = jnp.exp(lc)

        Skk = jnp.dot(K, Kt, preferred_element_type=F32, precision=HP)
        Sqk = jnp.dot(Q, Kt, preferred_element_type=F32, precision=HP)
        G = jnp.where(t_i > s_i, Skk * D, 0.0) * bc
        A = jnp.where(t_i >= s_i, Sqk * D, 0.0)

        Z = zall[h]
        R = (V - jnp.dot(K * gam, Z, preferred_element_type=F32,
                         precision=HP)) * bc
        U = R - jnp.dot(G, R, preferred_element_type=F32, precision=HP)
        Gp = G
        for _ in range(5):
            Gp = jnp.dot(Gp, Gp, preferred_element_type=F32, precision=HP)
            U = U + jnp.dot(Gp, U, preferred_element_type=F32, precision=HP)

        O = (jnp.dot(A, U, preferred_element_type=F32, precision=HP)
             + jnp.dot(Q * gam, Z, preferred_element_type=F32, precision=HP))
        o_parts.append(O)

        gC = jnp.exp(ell[C - 1:C, h:h + 1])
        dec_row = jnp.exp(ell[C - 1:C, h:h + 1] - lr)
        z_parts.append(Z * gC + jnp.dot(Kt * dec_row, U,
                                        preferred_element_type=F32,
                                        precision=HP))

    z_ref[...] = jnp.stack(z_parts, axis=0)
    o_full = jnp.concatenate(o_parts, axis=1)
    o_hi, o_lo = _split(o_full)
    out_ref[0] = _dot3(o_hi, o_lo, wouth_ref[...], woutl_ref[...])

    @pl.when(j == NC - 1)
    def _fin():
        for h in range(NUM_HEADS):
            sfin_ref[0, h] = z_ref[h].T


def kernel(x, in_proj_w, conv_w, conv_b, W_q, W_k, W_v, W_alpha, b_alpha,
           W_beta, b_beta, out_proj_w):
    Bsz, T, _ = x.shape
    winh, winl = _split(in_proj_w.T)
    convw2 = conv_w[:, 0, :].T
    convb2 = conv_b[None, :]
    wqh, wql = _split(W_q.T)
    wkh, wkl = _split(W_k.T)
    wvh, wvl = _split(W_v.T)
    wah, wal = _split(W_alpha.T)
    wbh, wbl = _split(W_beta.T)
    ba2, bb2 = b_alpha[None, :], b_beta[None, :]
    wouth, woutl = _split(out_proj_w.T)
    xh = x.reshape(Bsz, T // 8, 8, DIM)

    nblk = T // TB
    bw = lambda c, b2, i: (0, 0)
    H, dh = NUM_HEADS, HEAD_DIM
    wspec = pl.BlockSpec((DIM, D_INNER), bw)
    bspec = pl.BlockSpec((1, D_INNER), bw)
    q, k, v, al, be = pl.pallas_call(
        _proj_kernel,
        grid=(2, Bsz // 2, nblk),
        in_specs=[
            pl.BlockSpec((1, TB, DIM), lambda c, b2, i: (c * (Bsz // 2) + b2, i, 0)),
            pl.BlockSpec((1, 1, 8, DIM),
                         lambda c, b2, i: (c * (Bsz // 2) + b2,
                                           jnp.maximum(i * (TB // 8) - 1, 0), 0, 0)),
            wspec, wspec,
            pl.BlockSpec((D_CONV, D_INNER), bw),
            bspec,
            wspec, wspec, wspec, wspec, wspec, wspec,
            wspec, wspec, bspec, wspec, wspec, bspec,
        ],
        out_specs=[
            pl.BlockSpec((1, H, TB, dh), lambda c, b2, i: (c * (Bsz // 2) + b2, 0, i, 0)),
            pl.BlockSpec((1, H, TB, dh), lambda c, b2, i: (c * (Bsz // 2) + b2, 0, i, 0)),
            pl.BlockSpec((1, H, TB, dh), lambda c, b2, i: (c * (Bsz // 2) + b2, 0, i, 0)),
            pl.BlockSpec((1, TB, H), lambda c, b2, i: (c * (Bsz // 2) + b2, i, 0)),
            pl.BlockSpec((1, TB, H), lambda c, b2, i: (c * (Bsz // 2) + b2, i, 0)),
        ],
        out_shape=[
            jax.ShapeDtypeStruct((Bsz, H, T, dh), F32),
            jax.ShapeDtypeStruct((Bsz, H, T, dh), F32),
            jax.ShapeDtypeStruct((Bsz, H, T, dh), F32),
            jax.ShapeDtypeStruct((Bsz, T, H), F32),
            jax.ShapeDtypeStruct((Bsz, T, H), F32),
        ],
        compiler_params=pltpu.CompilerParams(
            dimension_semantics=("parallel", "arbitrary", "arbitrary"),
            vmem_limit_bytes=100 * 1024 * 1024,
        ),
        name="gdn_proj",
    )(x, xh, winh, winl, convw2, convb2, wqh, wql, wkh, wkl, wvh, wvl,
      wah, wal, ba2, wbh, wbl, bb2)

    nchunk = T // CHUNK
    out, sfin = pl.pallas_call(
        _delta_kernel,
        grid=(2, Bsz // 2, nchunk),
        in_specs=[
            pl.BlockSpec((1, H, CHUNK, dh), lambda c, b2, j: (c * (Bsz // 2) + b2, 0, j, 0)),
            pl.BlockSpec((1, H, CHUNK, dh), lambda c, b2, j: (c * (Bsz // 2) + b2, 0, j, 0)),
            pl.BlockSpec((1, H, CHUNK, dh), lambda c, b2, j: (c * (Bsz // 2) + b2, 0, j, 0)),
            pl.BlockSpec((1, CHUNK, H), lambda c, b2, j: (c * (Bsz // 2) + b2, j, 0)),
            pl.BlockSpec((1, CHUNK, H), lambda c, b2, j: (c * (Bsz // 2) + b2, j, 0)),
            pl.BlockSpec((D_INNER, DIM), bw),
            pl.BlockSpec((D_INNER, DIM), bw),
        ],
        out_specs=[
            pl.BlockSpec((1, CHUNK, DIM), lambda c, b2, j: (c * (Bsz // 2) + b2, j, 0)),
            pl.BlockSpec((1, H, dh, dh), lambda c, b2, j: (c * (Bsz // 2) + b2, 0, 0, 0)),
        ],
        out_shape=[
            jax.ShapeDtypeStruct((Bsz, T, DIM), F32),
            jax.ShapeDtypeStruct((Bsz, H, dh, dh), F32),
        ],
        scratch_shapes=[pltpu.VMEM((H, dh, dh), F32)],
        compiler_params=pltpu.CompilerParams(
            dimension_semantics=("parallel", "arbitrary", "arbitrary"),
            vmem_limit_bytes=100 * 1024 * 1024,
        ),
        name="gdn_delta",
    )(q, k, v, al, be, wouth, woutl)

    return out, sfin
